# baseline (speedup 1.0000x reference)
import functools

import jax
import jax.numpy as jnp
from jax import lax
from jax.experimental import pallas as pl
from jax.experimental.pallas import tpu as pltpu
from jax.experimental.pallas import tpu_sc as plsc

_ALPHA = 0.1
_NC = 2
_NS = 16
_NW = _NC * _NS
_C = 128


def _proj_body(tab_rows, e_ref, w_ref, b_ref, px_ref, py_ref, pw_ref, ph_ref):
    p = lax.dot_general(
        e_ref[...], w_ref[...],
        dimension_numbers=(((1,), (1,)), ((), ())),
        preferred_element_type=jnp.float32,
    ) * _ALPHA
    px_ref[...] = p[0:tab_rows]
    py_ref[...] = p[tab_rows:2 * tab_rows]
    pw_ref[...] = p[2 * tab_rows:3 * tab_rows]
    ph_ref[...] = p[3 * tab_rows:4 * tab_rows] + b_ref[...] * _ALPHA


def _project_tables(ecat, w_mlp, b2d, tab_rows):
    emb = ecat.shape[1]
    shp = jax.ShapeDtypeStruct((tab_rows, emb), jnp.float32)
    return pl.pallas_call(
        functools.partial(_proj_body, tab_rows),
        out_shape=[shp, shp, shp, shp],
    )(ecat, w_mlp, b2d)


def _sc_body(n_chunks, l_tiles, b_tiles,
             px_hbm, py_hbm, pw_hbm, ph_hbm,
             i0_hbm, i1_hbm, i2_hbm, i3_hbm, i4_hbm, i5_hbm, i6_hbm,
             eocr_hbm, out_hbm,
             idx0_v, idx1_v, rows0_v, rows1_v, tokrows0_v, tokrows1_v,
             outt_v,
             semi0, semi1, semg0, semg1, semo):
    wid = lax.axis_index("s") * _NC + lax.axis_index("c")
    per_w = n_chunks * _C
    idx_hbm = (i0_hbm, i1_hbm, i2_hbm, i3_hbm, i4_hbm, i5_hbm, i6_hbm)
    tabs = (px_hbm, py_hbm, px_hbm, py_hbm, pw_hbm, ph_hbm)
    idx_v = (idx0_v, idx1_v)
    rows_v = (rows0_v, rows1_v)
    tokrows_v = (tokrows0_v, tokrows1_v)
    semi = (semi0, semi1)
    semg = (semg0, semg1)
    ebase = lax.iota(jnp.int32, 16) * _C

    def fire_idx(t, s):
        base = wid * per_w + t * _C
        for j in range(7):
            pltpu.async_copy(idx_hbm[j].at[pl.ds(base, _C)],
                             idx_v[s].at[j], semi[s])

    def wait_idx(s):
        for j in range(7):
            pltpu.make_async_copy(
                idx_hbm[j].at[pl.ds(0, _C)], idx_v[s].at[j], semi[s]).wait()

    def fire_gathers(s):
        for j in range(6):
            pltpu.async_copy(
                tabs[j].at[idx_v[s].at[j]], rows_v[s].at[pl.ds(j * _C, _C)],
                semg[s])
        pltpu.async_copy(eocr_hbm.at[idx_v[s].at[6]], tokrows_v[s], semg[s])

    def wait_gathers(s):
        for j in range(6):
            pltpu.make_async_copy(
                tabs[j].at[idx_v[s].at[j]], rows_v[s].at[pl.ds(j * _C, _C)],
                semg[s]).wait()
        pltpu.make_async_copy(
            eocr_hbm.at[idx_v[s].at[6]], tokrows_v[s], semg[s]).wait()

    def fire_out(t):
        c = wid * n_chunks + t
        lt = c // (b_tiles * 8)
        rem = c % (b_tiles * 8)
        bt = rem // 8
        li = rem % 8
        ell = lt * 8 + li
        for et in range(8):
            off = ell * (8 * b_tiles * 8 * _C) + et * (b_tiles * 8 * _C) \
                + bt * (8 * _C)
            pltpu.async_copy(outt_v.at[pl.ds(et * 8 * _C, 8 * _C)],
                             out_hbm.at[pl.ds(off, 8 * _C)], semo)

    def wait_out():
        for et in range(8):
            pltpu.make_async_copy(
                outt_v.at[pl.ds(et * 8 * _C, 8 * _C)],
                out_hbm.at[pl.ds(0, 8 * _C)], semo).wait()

    def compute(s):
        rv = rows_v[s]
        tv = tokrows_v[s]

        def tok_body(i, carry2):
            for k in range(4):
                sl = pl.ds(k * 16, 16)
                acc = rv[i, sl]
                for j in range(1, 6):
                    acc = acc + rv[j * _C + i, sl]
                z = tv[i, sl] + jnp.maximum(acc, 0.0)
                plsc.store_scatter(outt_v, [ebase + (k * 16 * _C + i)], z)
            return carry2

        lax.fori_loop(0, _C, tok_body, 0)

    fire_idx(0, 0)
    fire_idx(1, 1)
    wait_idx(0)
    fire_gathers(0)

    def outer(c, carry):
        for b in range(2):
            t = c + b
            sn = 1 - b
            wait_gathers(b)

            @pl.when(t + 2 < n_chunks)
            def _():
                fire_idx(t + 2, b)

            @pl.when(t + 1 < n_chunks)
            def _():
                wait_idx(sn)
                fire_gathers(sn)

            @pl.when(t >= 1)
            def _():
                wait_out()

            compute(b)
            fire_out(t)
        return carry

    lax.fori_loop(0, n_chunks // 2, lambda c, carry: outer(2 * c, carry), 0)
    wait_out()


def _sc_lookup(tables, e_ocr, n, l_tiles, b_tiles, idx_flat):
    n_chunks = n // (_NW * _C)
    mesh = plsc.VectorSubcoreMesh(
        core_axis_name="c", subcore_axis_name="s",
        num_cores=_NC, num_subcores=_NS)
    emb = e_ocr.shape[1]
    f = pl.kernel(
        functools.partial(_sc_body, n_chunks, l_tiles, b_tiles),
        out_type=jax.ShapeDtypeStruct((n * emb,), jnp.float32),
        mesh=mesh,
        compiler_params=pltpu.CompilerParams(
            use_tc_tiling_on_sc=False, needs_layout_passes=False),
        scratch_types=[
            pltpu.VMEM((7, _C), jnp.int32),
            pltpu.VMEM((7, _C), jnp.int32),
            pltpu.VMEM((6 * _C, emb), jnp.float32),
            pltpu.VMEM((6 * _C, emb), jnp.float32),
            pltpu.VMEM((_C, emb), jnp.float32),
            pltpu.VMEM((_C, emb), jnp.float32),
            pltpu.VMEM((emb * _C,), jnp.float32),
            pltpu.SemaphoreType.DMA,
            pltpu.SemaphoreType.DMA,
            pltpu.SemaphoreType.DMA,
            pltpu.SemaphoreType.DMA,
            pltpu.SemaphoreType.DMA,
        ],
    )
    return f(*tables, *idx_flat, e_ocr)


def kernel(tok, x0, y0, x1, y1, w, h, E_ocr, E_x, E_y, E_w, E_h, W_mlp, b_mlp):
    b, l = tok.shape
    n = b * l
    emb = E_ocr.shape[1]
    tab_rows = E_x.shape[0]
    l_tiles = l // 8
    b_tiles = b // _C

    ecat = jnp.concatenate([E_x, E_y, E_w, E_h], axis=0)
    tables = _project_tables(ecat, W_mlp, b_mlp.reshape(1, -1), tab_rows)

    def phys(a):
        return (a.astype(jnp.int32).transpose(1, 0)
                .reshape(l_tiles, 8, b_tiles, _C)
                .transpose(0, 2, 1, 3)
                .reshape(-1))

    idx_flat = [phys(x0), phys(y0), phys(x1), phys(y1),
                phys(w), phys(h), phys(tok)]
    out = _sc_lookup(tables, E_ocr, n, l_tiles, b_tiles, idx_flat)

    return (out.reshape(l, emb // 8, b_tiles, 8, _C)
            .transpose(2, 4, 0, 1, 3)
            .reshape(b, l, emb))


# speedup vs baseline: 1.4885x; 1.4885x over previous
import functools

import jax
import jax.numpy as jnp
from jax import lax
from jax.experimental import pallas as pl
from jax.experimental.pallas import tpu as pltpu
from jax.experimental.pallas import tpu_sc as plsc

_ALPHA = 0.1
_NC = 2
_NS = 16
_NW = _NC * _NS
_C = 128


def _proj_body(tab_rows, e_ref, w_ref, b_ref, px_ref, py_ref, pw_ref, ph_ref):
    p = lax.dot_general(
        e_ref[...], w_ref[...],
        dimension_numbers=(((1,), (1,)), ((), ())),
        preferred_element_type=jnp.float32,
    ) * _ALPHA
    px_ref[...] = p[0:tab_rows]
    py_ref[...] = p[tab_rows:2 * tab_rows]
    pw_ref[...] = p[2 * tab_rows:3 * tab_rows]
    ph_ref[...] = p[3 * tab_rows:4 * tab_rows] + b_ref[...] * _ALPHA


def _project_tables(ecat, w_mlp, b2d, tab_rows):
    emb = ecat.shape[1]
    shp = jax.ShapeDtypeStruct((tab_rows, emb), jnp.float32)
    return pl.pallas_call(
        functools.partial(_proj_body, tab_rows),
        out_shape=[shp, shp, shp, shp],
    )(ecat, w_mlp, b2d)


def _sc_body(n_chunks, l_tiles, b_tiles,
             px_hbm, py_hbm, pw_hbm, ph_hbm,
             i0_hbm, i1_hbm, i2_hbm, i3_hbm, i4_hbm, i5_hbm, i6_hbm,
             eocr_hbm, out_hbm,
             idx0_v, idx1_v, rows0_v, rows1_v, tokrows0_v, tokrows1_v,
             outt_v,
             semi0, semi1, semg0, semg1, semo):
    wid = lax.axis_index("s") * _NC + lax.axis_index("c")
    per_w = n_chunks * _C
    idx_hbm = (i0_hbm, i1_hbm, i2_hbm, i3_hbm, i4_hbm, i5_hbm, i6_hbm)
    tabs = (px_hbm, py_hbm, px_hbm, py_hbm, pw_hbm, ph_hbm)
    idx_v = (idx0_v, idx1_v)
    rows_v = (rows0_v, rows1_v)
    tokrows_v = (tokrows0_v, tokrows1_v)
    semi = (semi0, semi1)
    semg = (semg0, semg1)
    ebase = lax.iota(jnp.int32, 16)

    def fire_idx(t, s):
        base = wid * per_w + t * _C
        for j in range(7):
            pltpu.async_copy(idx_hbm[j].at[pl.ds(base, _C)],
                             idx_v[s].at[j], semi[s])

    def wait_idx(s):
        for j in range(7):
            pltpu.make_async_copy(
                idx_hbm[j].at[pl.ds(0, _C)], idx_v[s].at[j], semi[s]).wait()

    def fire_gathers(s):
        for j in range(6):
            pltpu.async_copy(
                tabs[j].at[idx_v[s].at[j]], rows_v[s].at[pl.ds(j * _C, _C)],
                semg[s])
        pltpu.async_copy(eocr_hbm.at[idx_v[s].at[6]], tokrows_v[s], semg[s])

    def wait_gathers(s):
        for j in range(6):
            pltpu.make_async_copy(
                tabs[j].at[idx_v[s].at[j]], rows_v[s].at[pl.ds(j * _C, _C)],
                semg[s]).wait()
        pltpu.make_async_copy(
            eocr_hbm.at[idx_v[s].at[6]], tokrows_v[s], semg[s]).wait()

    def fire_out(t):
        c = wid * n_chunks + t
        lt = c // (b_tiles * 8)
        rem = c % (b_tiles * 8)
        bt = rem // 8
        li = rem % 8
        ell = lt * 8 + li
        for et in range(8):
            off = ell * (8 * b_tiles * 8) + et * (b_tiles * 8) + bt * 8
            pltpu.async_copy(
                outt_v.at[pl.ds(et * 8, 8), pl.ds(0, _C)],
                out_hbm.at[pl.ds(off, 8)], semo)

    def wait_out():
        for et in range(8):
            pltpu.make_async_copy(
                outt_v.at[pl.ds(et * 8, 8), pl.ds(0, _C)],
                out_hbm.at[pl.ds(0, 8)], semo).wait()

    def compute(s):
        rv = rows_v[s]
        tv = tokrows_v[s]

        def tok_body(i, carry2):
            for k in range(4):
                sl = pl.ds(k * 16, 16)
                acc = rv[i, sl]
                for j in range(1, 6):
                    acc = acc + rv[j * _C + i, sl]
                z = tv[i, sl] + jnp.maximum(acc, 0.0)
                plsc.store_scatter(outt_v, [k * 16 + ebase, ebase * 0 + i], z)
            return carry2

        lax.fori_loop(0, _C, tok_body, 0)

    fire_idx(0, 0)
    fire_idx(1, 1)
    wait_idx(0)
    fire_gathers(0)

    def outer(c, carry):
        for b in range(2):
            t = c + b
            sn = 1 - b
            wait_gathers(b)

            @pl.when(t + 2 < n_chunks)
            def _():
                fire_idx(t + 2, b)

            @pl.when(t + 1 < n_chunks)
            def _():
                wait_idx(sn)
                fire_gathers(sn)

            @pl.when(t >= 1)
            def _():
                wait_out()

            compute(b)
            fire_out(t)
        return carry

    lax.fori_loop(0, n_chunks // 2, lambda c, carry: outer(2 * c, carry), 0)
    wait_out()


def _sc_lookup(tables, e_ocr, n, l_tiles, b_tiles, idx_flat):
    n_chunks = n // (_NW * _C)
    mesh = plsc.VectorSubcoreMesh(
        core_axis_name="c", subcore_axis_name="s",
        num_cores=_NC, num_subcores=_NS)
    emb = e_ocr.shape[1]
    f = pl.kernel(
        functools.partial(_sc_body, n_chunks, l_tiles, b_tiles),
        out_type=jax.ShapeDtypeStruct((n * emb // _C, _C), jnp.float32),
        mesh=mesh,
        compiler_params=pltpu.CompilerParams(
            use_tc_tiling_on_sc=False, needs_layout_passes=False),
        scratch_types=[
            pltpu.VMEM((7, _C), jnp.int32),
            pltpu.VMEM((7, _C), jnp.int32),
            pltpu.VMEM((6 * _C, emb), jnp.float32),
            pltpu.VMEM((6 * _C, emb), jnp.float32),
            pltpu.VMEM((_C, emb), jnp.float32),
            pltpu.VMEM((_C, emb), jnp.float32),
            pltpu.VMEM((emb, _C + 1), jnp.float32),
            pltpu.SemaphoreType.DMA,
            pltpu.SemaphoreType.DMA,
            pltpu.SemaphoreType.DMA,
            pltpu.SemaphoreType.DMA,
            pltpu.SemaphoreType.DMA,
        ],
    )
    return f(*tables, *idx_flat, e_ocr)


def kernel(tok, x0, y0, x1, y1, w, h, E_ocr, E_x, E_y, E_w, E_h, W_mlp, b_mlp):
    b, l = tok.shape
    n = b * l
    emb = E_ocr.shape[1]
    tab_rows = E_x.shape[0]
    l_tiles = l // 8
    b_tiles = b // _C

    ecat = jnp.concatenate([E_x, E_y, E_w, E_h], axis=0)
    tables = _project_tables(ecat, W_mlp, b_mlp.reshape(1, -1), tab_rows)

    def phys(a):
        return (a.astype(jnp.int32).transpose(1, 0)
                .reshape(l_tiles, 8, b_tiles, _C)
                .transpose(0, 2, 1, 3)
                .reshape(-1))

    idx_flat = [phys(x0), phys(y0), phys(x1), phys(y1),
                phys(w), phys(h), phys(tok)]
    out = _sc_lookup(tables, E_ocr, n, l_tiles, b_tiles, idx_flat)

    return (out.reshape(l, emb // 8, b_tiles, 8, _C)
            .transpose(2, 4, 0, 1, 3)
            .reshape(b, l, emb))


# speedup vs baseline: 2.0857x; 1.4012x over previous
import functools

import jax
import jax.numpy as jnp
from jax import lax
from jax.experimental import pallas as pl
from jax.experimental.pallas import tpu as pltpu
from jax.experimental.pallas import tpu_sc as plsc

_ALPHA = 0.1
_NC = 2
_NS = 16
_NW = _NC * _NS
_C = 128


def _proj_body(tab_rows, e_ref, w_ref, b_ref, px_ref, py_ref, pw_ref, ph_ref):
    p = lax.dot_general(
        e_ref[...], w_ref[...],
        dimension_numbers=(((1,), (1,)), ((), ())),
        preferred_element_type=jnp.float32,
    ) * _ALPHA
    px_ref[...] = p[0:tab_rows]
    py_ref[...] = p[tab_rows:2 * tab_rows]
    pw_ref[...] = p[2 * tab_rows:3 * tab_rows]
    ph_ref[...] = p[3 * tab_rows:4 * tab_rows] + b_ref[...] * _ALPHA


def _project_tables(ecat, w_mlp, b2d, tab_rows):
    emb = ecat.shape[1]
    shp = jax.ShapeDtypeStruct((tab_rows, emb), jnp.float32)
    return pl.pallas_call(
        functools.partial(_proj_body, tab_rows),
        out_shape=[shp, shp, shp, shp],
    )(ecat, w_mlp, b2d)


def _sc_body(n_chunks, l_tiles, b_tiles,
             px_hbm, py_hbm, pw_hbm, ph_hbm,
             i0_hbm, i1_hbm, i2_hbm, i3_hbm, i4_hbm, i5_hbm, i6_hbm,
             eocr_hbm, out_hbm,
             idx0_v, idx1_v, acc0_v, acc1_v, tokrows0_v, tokrows1_v,
             outt_v,
             semi0, semi1, semf0, semf1, semg0, semg1, semo):
    wid = lax.axis_index("s") * _NC + lax.axis_index("c")
    per_w = n_chunks * _C
    idx_hbm = (i0_hbm, i1_hbm, i2_hbm, i3_hbm, i4_hbm, i5_hbm, i6_hbm)
    tabs = (px_hbm, py_hbm, px_hbm, py_hbm, pw_hbm, ph_hbm)
    idx_v = (idx0_v, idx1_v)
    acc_v = (acc0_v, acc1_v)
    tokrows_v = (tokrows0_v, tokrows1_v)
    semi = (semi0, semi1)
    semf = (semf0, semf1)
    semg = (semg0, semg1)
    ebase = lax.iota(jnp.int32, 16)

    def fire_idx(t, s):
        base = wid * per_w + t * _C
        for j in range(7):
            pltpu.async_copy(idx_hbm[j].at[pl.ds(base, _C)],
                             idx_v[s].at[j], semi[s])

    def wait_idx(s):
        for j in range(7):
            pltpu.make_async_copy(
                idx_hbm[j].at[pl.ds(0, _C)], idx_v[s].at[j], semi[s]).wait()

    def fire_first(s):
        pltpu.async_copy(tabs[0].at[idx_v[s].at[0]], acc_v[s], semf[s])
        pltpu.async_copy(eocr_hbm.at[idx_v[s].at[6]], tokrows_v[s], semg[s])

    def wait_first(s):
        pltpu.make_async_copy(
            tabs[0].at[idx_v[s].at[0]], acc_v[s], semf[s]).wait()

    def fire_rest(s):
        for j in range(1, 6):
            pltpu.async_copy(
                tabs[j].at[idx_v[s].at[j]], acc_v[s], semg[s], add=True)

    def wait_rest(s):
        for j in range(1, 6):
            pltpu.make_async_copy(
                tabs[j].at[idx_v[s].at[j]], acc_v[s], semg[s]).wait()
        pltpu.make_async_copy(
            eocr_hbm.at[idx_v[s].at[6]], tokrows_v[s], semg[s]).wait()

    def fire_out(t):
        c = wid * n_chunks + t
        lt = c // (b_tiles * 8)
        rem = c % (b_tiles * 8)
        bt = rem // 8
        li = rem % 8
        ell = lt * 8 + li
        for et in range(8):
            off = ell * (8 * b_tiles * 8) + et * (b_tiles * 8) + bt * 8
            pltpu.async_copy(
                outt_v.at[pl.ds(et * 8, 8), pl.ds(0, _C)],
                out_hbm.at[pl.ds(off, 8)], semo)

    def wait_out():
        for et in range(8):
            pltpu.make_async_copy(
                outt_v.at[pl.ds(et * 8, 8), pl.ds(0, _C)],
                out_hbm.at[pl.ds(0, 8)], semo).wait()

    def compute(s):
        rv = acc_v[s]
        tv = tokrows_v[s]

        def tok_body(i, carry2):
            for k in range(4):
                sl = pl.ds(k * 16, 16)
                z = tv[i, sl] + jnp.maximum(rv[i, sl], 0.0)
                plsc.store_scatter(outt_v, [k * 16 + ebase, ebase * 0 + i], z)
            return carry2

        lax.fori_loop(0, _C, tok_body, 0)

    fire_idx(0, 0)
    fire_idx(1, 1)
    wait_idx(0)
    fire_first(0)
    wait_first(0)
    fire_rest(0)

    def outer(c, carry):
        for b in range(2):
            t = c + b
            sn = 1 - b

            @pl.when(t + 1 < n_chunks)
            def _():
                wait_idx(sn)
                fire_first(sn)

            wait_rest(b)

            @pl.when(t + 2 < n_chunks)
            def _():
                fire_idx(t + 2, b)

            @pl.when(t >= 1)
            def _():
                wait_out()

            @pl.when(t + 1 < n_chunks)
            def _():
                wait_first(sn)
                fire_rest(sn)

            compute(b)
            fire_out(t)
        return carry

    lax.fori_loop(0, n_chunks // 2, lambda c, carry: outer(2 * c, carry), 0)
    wait_out()


def _sc_lookup(tables, e_ocr, n, l_tiles, b_tiles, idx_flat):
    n_chunks = n // (_NW * _C)
    mesh = plsc.VectorSubcoreMesh(
        core_axis_name="c", subcore_axis_name="s",
        num_cores=_NC, num_subcores=_NS)
    emb = e_ocr.shape[1]
    f = pl.kernel(
        functools.partial(_sc_body, n_chunks, l_tiles, b_tiles),
        out_type=jax.ShapeDtypeStruct((n * emb // _C, _C), jnp.float32),
        mesh=mesh,
        compiler_params=pltpu.CompilerParams(
            use_tc_tiling_on_sc=False, needs_layout_passes=False),
        scratch_types=[
            pltpu.VMEM((7, _C), jnp.int32),
            pltpu.VMEM((7, _C), jnp.int32),
            pltpu.VMEM((_C, emb), jnp.float32),
            pltpu.VMEM((_C, emb), jnp.float32),
            pltpu.VMEM((_C, emb), jnp.float32),
            pltpu.VMEM((_C, emb), jnp.float32),
            pltpu.VMEM((emb, _C + 1), jnp.float32),
            pltpu.SemaphoreType.DMA,
            pltpu.SemaphoreType.DMA,
            pltpu.SemaphoreType.DMA,
            pltpu.SemaphoreType.DMA,
            pltpu.SemaphoreType.DMA,
            pltpu.SemaphoreType.DMA,
            pltpu.SemaphoreType.DMA,
        ],
    )
    return f(*tables, *idx_flat, e_ocr)


def kernel(tok, x0, y0, x1, y1, w, h, E_ocr, E_x, E_y, E_w, E_h, W_mlp, b_mlp):
    b, l = tok.shape
    n = b * l
    emb = E_ocr.shape[1]
    tab_rows = E_x.shape[0]
    l_tiles = l // 8
    b_tiles = b // _C

    ecat = jnp.concatenate([E_x, E_y, E_w, E_h], axis=0)
    tables = _project_tables(ecat, W_mlp, b_mlp.reshape(1, -1), tab_rows)

    def phys(a):
        return (a.astype(jnp.int32).transpose(1, 0)
                .reshape(l_tiles, 8, b_tiles, _C)
                .transpose(0, 2, 1, 3)
                .reshape(-1))

    idx_flat = [phys(x0), phys(y0), phys(x1), phys(y1),
                phys(w), phys(h), phys(tok)]
    out = _sc_lookup(tables, E_ocr, n, l_tiles, b_tiles, idx_flat)

    return (out.reshape(l, emb // 8, b_tiles, 8, _C)
            .transpose(2, 4, 0, 1, 3)
            .reshape(b, l, emb))


# speedup vs baseline: 2.1090x; 1.0112x over previous
import functools

import jax
import jax.numpy as jnp
from jax import lax
from jax.experimental import pallas as pl
from jax.experimental.pallas import tpu as pltpu
from jax.experimental.pallas import tpu_sc as plsc

_ALPHA = 0.1
_NC = 2
_NS = 16
_NW = _NC * _NS
_C = 128


def _proj_body(tab_rows, e_ref, w_ref, b_ref, px_ref, py_ref, pw_ref, ph_ref):
    p = lax.dot_general(
        e_ref[...], w_ref[...],
        dimension_numbers=(((1,), (1,)), ((), ())),
        preferred_element_type=jnp.float32,
    ) * _ALPHA
    px_ref[...] = p[0:tab_rows]
    py_ref[...] = p[tab_rows:2 * tab_rows]
    pw_ref[...] = p[2 * tab_rows:3 * tab_rows]
    ph_ref[...] = p[3 * tab_rows:4 * tab_rows] + b_ref[...] * _ALPHA


def _project_tables(ecat, w_mlp, b2d, tab_rows):
    emb = ecat.shape[1]
    shp = jax.ShapeDtypeStruct((tab_rows, emb), jnp.float32)
    return pl.pallas_call(
        functools.partial(_proj_body, tab_rows),
        out_shape=[shp, shp, shp, shp],
    )(ecat, w_mlp, b2d)


def _sc_body(n_chunks, l_tiles, b_tiles,
             px_hbm, py_hbm, pw_hbm, ph_hbm,
             i0_hbm, i1_hbm, i2_hbm, i3_hbm, i4_hbm, i5_hbm, i6_hbm,
             eocr_hbm, out_hbm,
             idx0_v, idx1_v, acc0_v, acc1_v, tokrows0_v, tokrows1_v,
             outt_v,
             semi0, semi1, semf0, semf1, semg0, semg1, semo):
    wid = lax.axis_index("s") * _NC + lax.axis_index("c")
    per_w = n_chunks * _C
    idx_hbm = (i0_hbm, i1_hbm, i2_hbm, i3_hbm, i4_hbm, i5_hbm, i6_hbm)
    tabs = (px_hbm, py_hbm, px_hbm, py_hbm, pw_hbm, ph_hbm)
    idx_v = (idx0_v, idx1_v)
    acc_v = (acc0_v, acc1_v)
    tokrows_v = (tokrows0_v, tokrows1_v)
    semi = (semi0, semi1)
    semf = (semf0, semf1)
    semg = (semg0, semg1)
    ebase = lax.iota(jnp.int32, 16)

    def fire_idx(t, s):
        base = wid * per_w + t * _C
        for j in range(7):
            pltpu.async_copy(idx_hbm[j].at[pl.ds(base, _C)],
                             idx_v[s].at[j], semi[s])

    def wait_idx(s):
        for j in range(7):
            pltpu.make_async_copy(
                idx_hbm[j].at[pl.ds(0, _C)], idx_v[s].at[j], semi[s]).wait()

    def fire_first(s):
        pltpu.async_copy(tabs[0].at[idx_v[s].at[0]], acc_v[s], semf[s])
        pltpu.async_copy(eocr_hbm.at[idx_v[s].at[6]], tokrows_v[s], semg[s])

    def wait_first(s):
        pltpu.make_async_copy(
            tabs[0].at[idx_v[s].at[0]], acc_v[s], semf[s]).wait()

    def fire_rest(s):
        for j in range(1, 6):
            pltpu.async_copy(
                tabs[j].at[idx_v[s].at[j]], acc_v[s], semg[s], add=True)

    def wait_rest(s):
        for j in range(1, 6):
            pltpu.make_async_copy(
                tabs[j].at[idx_v[s].at[j]], acc_v[s], semg[s]).wait()
        pltpu.make_async_copy(
            eocr_hbm.at[idx_v[s].at[6]], tokrows_v[s], semg[s]).wait()

    def fire_out(t):
        c = wid * n_chunks + t
        lt = c // (b_tiles * 8)
        rem = c % (b_tiles * 8)
        bt = rem // 8
        li = rem % 8
        ell = lt * 8 + li
        for et in range(8):
            off = ell * (8 * b_tiles * 8) + et * (b_tiles * 8) + bt * 8
            pltpu.async_copy(
                outt_v.at[pl.ds(et * 8, 8), pl.ds(0, _C)],
                out_hbm.at[pl.ds(off, 8)], semo)

    def wait_out():
        for et in range(8):
            pltpu.make_async_copy(
                outt_v.at[pl.ds(et * 8, 8), pl.ds(0, _C)],
                out_hbm.at[pl.ds(0, 8)], semo).wait()

    def compute(s):
        rv = acc_v[s]
        tv = tokrows_v[s]

        def tok_body(i, carry2):
            for k in range(4):
                sl = pl.ds(k * 16, 16)
                z = tv[i, sl] + jnp.maximum(rv[i, sl], 0.0)
                plsc.store_scatter(outt_v, [k * 16 + ebase, ebase * 0 + i], z)
            return carry2

        lax.fori_loop(0, _C, tok_body, 0, unroll=4)

    fire_idx(0, 0)
    fire_idx(1, 1)
    wait_idx(0)
    fire_first(0)
    wait_first(0)
    fire_rest(0)

    def outer(c, carry):
        for b in range(2):
            t = c + b
            sn = 1 - b

            @pl.when(t + 1 < n_chunks)
            def _():
                wait_idx(sn)
                fire_first(sn)

            wait_rest(b)

            @pl.when(t + 2 < n_chunks)
            def _():
                fire_idx(t + 2, b)

            @pl.when(t >= 1)
            def _():
                wait_out()

            @pl.when(t + 1 < n_chunks)
            def _():
                wait_first(sn)
                fire_rest(sn)

            compute(b)
            fire_out(t)
        return carry

    lax.fori_loop(0, n_chunks // 2, lambda c, carry: outer(2 * c, carry), 0)
    wait_out()


def _sc_lookup(tables, e_ocr, n, l_tiles, b_tiles, idx_flat):
    n_chunks = n // (_NW * _C)
    mesh = plsc.VectorSubcoreMesh(
        core_axis_name="c", subcore_axis_name="s",
        num_cores=_NC, num_subcores=_NS)
    emb = e_ocr.shape[1]
    f = pl.kernel(
        functools.partial(_sc_body, n_chunks, l_tiles, b_tiles),
        out_type=jax.ShapeDtypeStruct((n * emb // _C, _C), jnp.float32),
        mesh=mesh,
        compiler_params=pltpu.CompilerParams(
            use_tc_tiling_on_sc=False, needs_layout_passes=False),
        scratch_types=[
            pltpu.VMEM((7, _C), jnp.int32),
            pltpu.VMEM((7, _C), jnp.int32),
            pltpu.VMEM((_C, emb), jnp.float32),
            pltpu.VMEM((_C, emb), jnp.float32),
            pltpu.VMEM((_C, emb), jnp.float32),
            pltpu.VMEM((_C, emb), jnp.float32),
            pltpu.VMEM((emb, _C + 1), jnp.float32),
            pltpu.SemaphoreType.DMA,
            pltpu.SemaphoreType.DMA,
            pltpu.SemaphoreType.DMA,
            pltpu.SemaphoreType.DMA,
            pltpu.SemaphoreType.DMA,
            pltpu.SemaphoreType.DMA,
            pltpu.SemaphoreType.DMA,
        ],
    )
    return f(*tables, *idx_flat, e_ocr)


def kernel(tok, x0, y0, x1, y1, w, h, E_ocr, E_x, E_y, E_w, E_h, W_mlp, b_mlp):
    b, l = tok.shape
    n = b * l
    emb = E_ocr.shape[1]
    tab_rows = E_x.shape[0]
    l_tiles = l // 8
    b_tiles = b // _C

    ecat = jnp.concatenate([E_x, E_y, E_w, E_h], axis=0)
    tables = _project_tables(ecat, W_mlp, b_mlp.reshape(1, -1), tab_rows)

    def phys(a):
        return (a.astype(jnp.int32).transpose(1, 0)
                .reshape(l_tiles, 8, b_tiles, _C)
                .transpose(0, 2, 1, 3)
                .reshape(-1))

    idx_flat = [phys(x0), phys(y0), phys(x1), phys(y1),
                phys(w), phys(h), phys(tok)]
    out = _sc_lookup(tables, E_ocr, n, l_tiles, b_tiles, idx_flat)

    return (out.reshape(l, emb // 8, b_tiles, 8, _C)
            .transpose(2, 4, 0, 1, 3)
            .reshape(b, l, emb))
